# Initial kernel scaffold; baseline (speedup 1.0000x reference)
import jax
import jax.numpy as jnp
from jax.experimental import pallas as pl


def kernel(indices, cluster_usage, embedding_sum):
    raise NotImplementedError("write your pallas kernel here")


# speedup vs baseline: 3.7218x; 3.7218x over previous
import functools

import jax
import jax.numpy as jnp
from jax import lax
from jax.experimental import pallas as pl
from jax.experimental.pallas import tpu as pltpu
from jax.experimental.pallas import tpu_sc as plsc

_K = 8192
_D = 256
_CHUNK = 128


def _div_body(usage_ref, sum_ref, out_ref):
    out_ref[...] = sum_ref[...] / usage_ref[...]


def _normalized_table(cluster_usage, embedding_sum):
    K, D = embedding_sum.shape
    return pl.pallas_call(
        _div_body,
        out_shape=jax.ShapeDtypeStruct((K, D), jnp.float32),
    )(cluster_usage[:, None], embedding_sum)


@functools.cache
def _make_gather(N, D, NC, NS):
    NW = NC * NS
    per_w = N // NW
    nch = per_w // _CHUNK
    mesh = plsc.VectorSubcoreMesh(core_axis_name="c", subcore_axis_name="s")

    @functools.partial(
        pl.kernel,
        mesh=mesh,
        out_type=jax.ShapeDtypeStruct((N, D), jnp.float32),
        scratch_types=[
            pltpu.VMEM((nch, _CHUNK), jnp.int32),
            pltpu.VMEM((2, _CHUNK, D), jnp.float32),
            pltpu.SemaphoreType.DMA,
            pltpu.SemaphoreType.DMA,
            pltpu.SemaphoreType.DMA,
            pltpu.SemaphoreType.DMA,
        ],
    )
    def gather(table_hbm, idx_hbm, out_hbm, idx_v, rows_v, g0, g1, s0, s1):
        wid = lax.axis_index("s") * NC + lax.axis_index("c")
        base = wid * per_w
        gsem = (g0, g1)
        ssem = (s0, s1)

        pltpu.sync_copy(idx_hbm.at[wid], idx_v)

        gathers = [None, None]
        scatters = [None, None]

        def start_gather(c):
            b = c % 2
            gathers[b] = pltpu.make_async_copy(
                table_hbm.at[idx_v.at[c]], rows_v.at[b], gsem[b])
            gathers[b].start()

        start_gather(0)
        for c in range(nch):
            b = c % 2
            if c + 1 < nch:
                bn = (c + 1) % 2
                if scatters[bn] is not None:
                    scatters[bn].wait()
                start_gather(c + 1)
            gathers[b].wait()
            scatters[b] = pltpu.make_async_copy(
                rows_v.at[b], out_hbm.at[pl.ds(base + c * _CHUNK, _CHUNK)],
                ssem[b])
            scatters[b].start()
        for b in range(2):
            if scatters[b] is not None:
                scatters[b].wait()

    return gather


def kernel(indices, cluster_usage, embedding_sum):
    K, D = embedding_sum.shape
    B, T = indices.shape
    N = B * T

    info = plsc.get_sparse_core_info()
    NC, NS = info.num_cores, info.num_subcores
    NW = NC * NS
    assert N % (NW * _CHUNK) == 0

    table = _normalized_table(cluster_usage, embedding_sum)
    idx = indices.astype(jnp.int32).reshape(NW, N // (NW * _CHUNK), _CHUNK)
    out = _make_gather(N, D, NC, NS)(table, idx)
    return out.reshape(B, T, D)
